# speedup vs baseline: 4.8091x; 4.8091x over previous
import functools

import jax
import jax.numpy as jnp
from jax import lax
from jax.experimental import pallas as pl
from jax.experimental.pallas import tpu as pltpu
from jax.experimental.pallas import tpu_sc as plsc

N = 10000
E = 320000
D = 128
G = 32
C = 10
NPAD = 10240
NWORKERS = 32
EPT = E // NWORKERS
CH = 80
NCHUNK = EPT // CH
ROWS_PER_TILE = NPAD // 16


def _sc_agg_body(src_hbm, dst_hbm, table_hbm, zeros_hbm, out_hbm,
                 isrc, idst, rows, acc, sem):
    cid = lax.axis_index("c")
    sid = lax.axis_index("s")
    wid = cid * 16 + sid
    base = wid * EPT

    pltpu.sync_copy(zeros_hbm, acc.at[pl.ds(sid * ROWS_PER_TILE, ROWS_PER_TILE)])
    plsc.subcore_barrier()

    def chunk(c, carry):
        off = base + c * CH
        pltpu.sync_copy(src_hbm.at[pl.ds(off, CH)], isrc)
        pltpu.sync_copy(dst_hbm.at[pl.ds(off, CH)], idst)
        pltpu.async_copy(table_hbm.at[isrc], rows, sem).wait()
        pltpu.sync_copy(rows, acc.at[idst], add=True)
        return carry

    lax.fori_loop(0, NCHUNK, chunk, 0)
    plsc.subcore_barrier()

    pltpu.sync_copy(acc.at[pl.ds(sid * ROWS_PER_TILE, ROWS_PER_TILE)],
                    out_hbm.at[cid, pl.ds(sid * ROWS_PER_TILE, ROWS_PER_TILE)])


@jax.jit
def _sc_agg(src, dst, table, zeros_blk):
    mesh = plsc.VectorSubcoreMesh(core_axis_name="c", subcore_axis_name="s")
    return pl.kernel(
        _sc_agg_body,
        out_type=jax.ShapeDtypeStruct((2, NPAD, D), jnp.float32),
        mesh=mesh,
        scratch_types=[
            pltpu.VMEM((CH,), jnp.int32),
            pltpu.VMEM((CH,), jnp.int32),
            pltpu.VMEM((CH, D), jnp.float32),
            pltpu.VMEM_SHARED((NPAD, D), jnp.float32),
            pltpu.SemaphoreType.DMA,
        ],
    )(src, dst, table, zeros_blk)


def _mlp1_body(x_ref, agg_ref, w1_ref, b1_ref, w2_ref, b2_ref, eps_ref, o_ref):
    a = agg_ref[0] + agg_ref[1]
    xt = x_ref[...] * eps_ref[...] + a
    h1 = jnp.maximum(
        jnp.dot(xt, w1_ref[...], preferred_element_type=jnp.float32)
        + b1_ref[...], 0.0)
    h2 = jnp.dot(h1, w2_ref[...], preferred_element_type=jnp.float32) + b2_ref[...]
    o_ref[...] = jnp.maximum(h2, 0.0)


@jax.jit
def _mlp1(xp, aggs, W1, b1, W2, b2, epsv):
    return pl.pallas_call(
        _mlp1_body,
        out_shape=jax.ShapeDtypeStruct((NPAD, D), jnp.float32),
    )(xp, aggs, W1, b1, W2, b2, epsv)


def _mlp2_pool_body(h_ref, agg_ref, w1_ref, b1_ref, w2_ref, b2_ref, eps_ref,
                    batch_ref, wl_ref, bl_ref, o_ref):
    a = agg_ref[0] + agg_ref[1]
    xt = h_ref[...] * eps_ref[...] + a
    h1 = jnp.maximum(
        jnp.dot(xt, w1_ref[...], preferred_element_type=jnp.float32)
        + b1_ref[...], 0.0)
    h2 = jnp.dot(h1, w2_ref[...], preferred_element_type=jnp.float32) + b2_ref[...]
    gids = lax.broadcasted_iota(jnp.int32, (G, NPAD), 0)
    onehot = (batch_ref[...] == gids).astype(jnp.float32)
    sums = jnp.dot(onehot, h2, preferred_element_type=jnp.float32)
    cnt = jnp.sum(onehot, axis=1, keepdims=True)
    pooled = sums / jnp.maximum(cnt, 1.0)
    logits = jnp.dot(pooled, wl_ref[...], preferred_element_type=jnp.float32) \
        + bl_ref[...]
    m = jnp.max(logits, axis=-1, keepdims=True)
    lse = m + jnp.log(jnp.sum(jnp.exp(logits - m), axis=-1, keepdims=True))
    o_ref[...] = logits - lse


@jax.jit
def _mlp2_pool(h, aggs, W1, b1, W2, b2, epsv, batch_r, Wl, bl):
    return pl.pallas_call(
        _mlp2_pool_body,
        out_shape=jax.ShapeDtypeStruct((G, C), jnp.float32),
    )(h, aggs, W1, b1, W2, b2, epsv, batch_r, Wl, bl)


def kernel(x, edge_index, batch, eps1, W11, b11, W12, b12,
           eps2, W21, b21, W22, b22, Wl, bl):
    xp = jnp.zeros((NPAD, D), jnp.float32).at[:N].set(x)
    src = edge_index[0]
    dst = edge_index[1]
    batch_r = jnp.full((1, NPAD), -1, jnp.int32).at[0, :N].set(batch)
    zeros_blk = jnp.zeros((ROWS_PER_TILE, D), jnp.float32)
    eps1v = jnp.broadcast_to(jnp.reshape(1.0 + eps1, (1, 1)), (1, D))
    eps2v = jnp.broadcast_to(jnp.reshape(1.0 + eps2, (1, 1)), (1, D))
    b11r = jnp.reshape(b11, (1, D))
    b12r = jnp.reshape(b12, (1, D))
    b21r = jnp.reshape(b21, (1, D))
    b22r = jnp.reshape(b22, (1, D))
    blr = jnp.reshape(bl, (1, C))

    aggs1 = _sc_agg(src, dst, xp, zeros_blk)
    h = _mlp1(xp, aggs1, W11, b11r, W12, b12r, eps1v)
    aggs2 = _sc_agg(src, dst, h, zeros_blk)
    return _mlp2_pool(h, aggs2, W21, b21r, W22, b22r, eps2v, batch_r, Wl, blr)


# speedup vs baseline: 10.1939x; 2.1197x over previous
import functools

import jax
import jax.numpy as jnp
from jax import lax
from jax.experimental import pallas as pl
from jax.experimental.pallas import tpu as pltpu
from jax.experimental.pallas import tpu_sc as plsc

N = 10000
E = 320000
D = 128
G = 32
C = 10
NPAD = 10240
NWORKERS = 32
EPT = E // NWORKERS
CH = 80
NCHUNK = EPT // CH
SUP = 25
NSUP = NCHUNK // SUP
ROWS_PER_TILE = NPAD // 16


def _sc_agg_body(src_hbm, dst_hbm, table_hbm, zeros_hbm, out_hbm,
                 isrc, idst, rows0, rows1, acc, sem0, sem1):
    cid = lax.axis_index("c")
    sid = lax.axis_index("s")
    wid = cid * 16 + sid

    pltpu.sync_copy(zeros_hbm, acc.at[pl.ds(sid * ROWS_PER_TILE, ROWS_PER_TILE)])

    plsc.subcore_barrier()

    def gstart(c, buf, sem):
        cc = jnp.minimum(c, SUP - 1)
        pltpu.async_copy(table_hbm.at[isrc.at[cc]], buf, sem)

    def gwait(buf, sem):
        pltpu.make_async_copy(table_hbm.at[isrc.at[0]], buf, sem).wait()

    def scat(c, buf):
        pltpu.sync_copy(buf, acc.at[idst.at[c]], add=True)

    def superchunk(s, carry):
        pltpu.sync_copy(src_hbm.at[wid, s], isrc)
        pltpu.sync_copy(dst_hbm.at[wid, s], idst)

        gstart(0, rows0, sem0)
        gstart(1, rows1, sem1)

        def body(i, carry):
            c0 = 2 * i
            gwait(rows0, sem0)
            scat(c0, rows0)
            gstart(c0 + 2, rows0, sem0)
            gwait(rows1, sem1)
            scat(c0 + 1, rows1)
            gstart(c0 + 3, rows1, sem1)
            return carry

        lax.fori_loop(0, (SUP - 1) // 2, body, 0)
        gwait(rows0, sem0)
        scat(SUP - 1, rows0)
        gwait(rows1, sem1)
        return carry

    lax.fori_loop(0, NSUP, superchunk, 0)

    plsc.subcore_barrier()
    pltpu.sync_copy(acc.at[pl.ds(sid * ROWS_PER_TILE, ROWS_PER_TILE)],
                    out_hbm.at[cid, pl.ds(sid * ROWS_PER_TILE, ROWS_PER_TILE)])


@jax.jit
def _sc_agg(src, dst, table, zeros_blk):
    mesh = plsc.VectorSubcoreMesh(core_axis_name="c", subcore_axis_name="s")
    return pl.kernel(
        _sc_agg_body,
        out_type=jax.ShapeDtypeStruct((2, NPAD, D), jnp.float32),
        mesh=mesh,
        scratch_types=[
            pltpu.VMEM((SUP, CH), jnp.int32),
            pltpu.VMEM((SUP, CH), jnp.int32),
            pltpu.VMEM((CH, D), jnp.float32),
            pltpu.VMEM((CH, D), jnp.float32),
            pltpu.VMEM_SHARED((NPAD, D), jnp.float32),
            pltpu.SemaphoreType.DMA,
            pltpu.SemaphoreType.DMA,
        ],
    )(src, dst, table, zeros_blk)


def _mlp1_body(x_ref, agg_ref, w1_ref, b1_ref, w2_ref, b2_ref, eps_ref, o_ref):
    a = agg_ref[0] + agg_ref[1]
    xt = x_ref[...] * eps_ref[...] + a
    h1 = jnp.maximum(
        jnp.dot(xt, w1_ref[...], preferred_element_type=jnp.float32)
        + b1_ref[...], 0.0)
    h2 = jnp.dot(h1, w2_ref[...], preferred_element_type=jnp.float32) + b2_ref[...]
    o_ref[...] = jnp.maximum(h2, 0.0)


@jax.jit
def _mlp1(xp, aggs, W1, b1, W2, b2, epsv):
    return pl.pallas_call(
        _mlp1_body,
        out_shape=jax.ShapeDtypeStruct((NPAD, D), jnp.float32),
    )(xp, aggs, W1, b1, W2, b2, epsv)


def _mlp2_pool_body(h_ref, agg_ref, w1_ref, b1_ref, w2_ref, b2_ref, eps_ref,
                    batch_ref, wl_ref, bl_ref, o_ref):
    a = agg_ref[0] + agg_ref[1]
    xt = h_ref[...] * eps_ref[...] + a
    h1 = jnp.maximum(
        jnp.dot(xt, w1_ref[...], preferred_element_type=jnp.float32)
        + b1_ref[...], 0.0)
    h2 = jnp.dot(h1, w2_ref[...], preferred_element_type=jnp.float32) + b2_ref[...]
    gids = lax.broadcasted_iota(jnp.int32, (G, NPAD), 0)
    onehot = (batch_ref[...] == gids).astype(jnp.float32)
    sums = jnp.dot(onehot, h2, preferred_element_type=jnp.float32)
    cnt = jnp.sum(onehot, axis=1, keepdims=True)
    pooled = sums / jnp.maximum(cnt, 1.0)
    logits = jnp.dot(pooled, wl_ref[...], preferred_element_type=jnp.float32) \
        + bl_ref[...]
    m = jnp.max(logits, axis=-1, keepdims=True)
    lse = m + jnp.log(jnp.sum(jnp.exp(logits - m), axis=-1, keepdims=True))
    o_ref[...] = logits - lse


@jax.jit
def _mlp2_pool(h, aggs, W1, b1, W2, b2, epsv, batch_r, Wl, bl):
    return pl.pallas_call(
        _mlp2_pool_body,
        out_shape=jax.ShapeDtypeStruct((G, C), jnp.float32),
    )(h, aggs, W1, b1, W2, b2, epsv, batch_r, Wl, bl)


def kernel(x, edge_index, batch, eps1, W11, b11, W12, b12,
           eps2, W21, b21, W22, b22, Wl, bl):
    xp = jnp.zeros((NPAD, D), jnp.float32).at[:N].set(x)
    src = jnp.reshape(edge_index[0], (NWORKERS, NSUP, SUP, CH))
    dst = jnp.reshape(edge_index[1], (NWORKERS, NSUP, SUP, CH))
    batch_r = jnp.full((1, NPAD), -1, jnp.int32).at[0, :N].set(batch)
    zeros_blk = jnp.zeros((ROWS_PER_TILE, D), jnp.float32)
    eps1v = jnp.broadcast_to(jnp.reshape(1.0 + eps1, (1, 1)), (1, D))
    eps2v = jnp.broadcast_to(jnp.reshape(1.0 + eps2, (1, 1)), (1, D))
    b11r = jnp.reshape(b11, (1, D))
    b12r = jnp.reshape(b12, (1, D))
    b21r = jnp.reshape(b21, (1, D))
    b22r = jnp.reshape(b22, (1, D))
    blr = jnp.reshape(bl, (1, C))

    aggs1 = _sc_agg(src, dst, xp, zeros_blk)
    h = _mlp1(xp, aggs1, W11, b11r, W12, b12r, eps1v)
    aggs2 = _sc_agg(src, dst, h, zeros_blk)
    return _mlp2_pool(h, aggs2, W21, b21r, W22, b22r, eps2v, batch_r, Wl, blr)
